# Initial kernel scaffold; baseline (speedup 1.0000x reference)
import jax
import jax.numpy as jnp
from jax.experimental import pallas as pl


def kernel(low_context_vector, original_node, mask, id, low_init_w, W_ctx, b_ctx, W_v, b_v, W_t, b_t, W_q, b_q, v_ptr):
    raise NotImplementedError("write your pallas kernel here")


# speedup vs baseline: 2.2685x; 2.2685x over previous
import jax
import jax.numpy as jnp
from jax.experimental import pallas as pl
from jax.experimental.pallas import tpu as pltpu

_B, _S, _D, _H = 128, 32, 128, 128
_C = 10.0
_NEG = -jnp.inf


def _decoder_kernel(f0_ref, lcv_ref, nodex_ref, nodey_ref, mask_ref, gum_ref,
                    liw_ref, Wc_ref, bc_ref, Wv_ref, bv_ref,
                    Wt_ref, bt_ref, Wq_ref, bq_ref, vp_ref,
                    logp_ref, idx_ref, last_ref, R_ref):
    lcv = lcv_ref[:]
    Wv = Wv_ref[:]
    Wq = Wq_ref[:]
    vp = vp_ref[:]
    bq = bq_ref[:]
    bv = bv_ref[:]

    lcv2 = lcv.reshape(_B * _S, _D)
    T = (jnp.dot(lcv2, Wt_ref[:].T) + bt_ref[:]).reshape(_B, _S, _H)
    h_bar = jnp.dot(jnp.mean(lcv, axis=1), Wc_ref[:].T) + bc_ref[:]
    q0 = h_bar + (jnp.dot(liw_ref[:], Wv.T) + bv)

    col = jax.lax.broadcasted_iota(jnp.int32, (_B, _S), 1)
    colf = jax.lax.broadcasted_iota(jnp.int32, (_B, _S), 1)

    def attend_sample(q, mask, g):
        qh = jnp.dot(q, Wq.T) + bq
        u = jnp.tanh(T + qh[:, None, :])
        lg = _C * jnp.tanh(jnp.dot(u.reshape(_B * _S, _H), vp).reshape(_B, _S))
        lg = jnp.where(mask == 1.0, _NEG, lg)
        shifted = lg - jnp.max(lg, axis=1, keepdims=True)
        logp = shifted - jnp.log(jnp.sum(jnp.exp(shifted), axis=1, keepdims=True))
        score = lg + g
        smax = jnp.max(score, axis=1, keepdims=True)
        idx = jnp.min(jnp.where(score == smax, col, _S), axis=1)
        return idx, logp

    def gather_update(idx, logp, mask, q_new_parts):
        onehot = col == idx[:, None]
        step_logp = jnp.sum(jnp.where(onehot, logp, 0.0), axis=1)
        mask = jnp.where(onehot, 1.0, mask)
        ohf = jnp.where(onehot, 1.0, 0.0)
        low_h = jnp.sum(lcv * ohf[:, :, None], axis=1)
        nx = jnp.sum(jnp.where(onehot, nodex_ref[:], 0.0), axis=1)
        ny = jnp.sum(jnp.where(onehot, nodey_ref[:], 0.0), axis=1)
        return onehot, step_logp, mask, low_h, nx, ny

    mask = mask_ref[:]
    g0 = gum_ref[0]
    idx0, logp0 = attend_sample(q0, mask, g0)
    idx0 = jnp.where(f0_ref[0] == 1, jnp.zeros_like(idx0), idx0)
    _, slp0, mask, ih, nx0, ny0 = gather_update(idx0, logp0, mask, None)
    cat0 = jnp.concatenate([ih, ih], axis=1)
    q = h_bar + (jnp.dot(cat0, Wv.T) + bv)
    cx, cy = nodex_ref[:, 0], nodey_ref[:, 0]
    dx0, dy0 = nx0 - cx, ny0 - cy
    r0 = jnp.sqrt(dx0 * dx0 + dy0 * dy0)

    logp_acc = jnp.where(colf == 0, slp0[:, None], 0.0)
    idx_acc = jnp.where(colf == 0, idx0[:, None], 0)
    R_acc = jnp.where(colf == 0, r0[:, None], 0.0)

    def body(i, carry):
        q, mask, cx, cy, logp_acc, idx_acc, R_acc = carry
        g = gum_ref[i]
        idx, logp = attend_sample(q, mask, g)
        onehot, slp, mask, low_h, nx, ny = gather_update(idx, logp, mask, None)
        cat = jnp.concatenate([ih, low_h], axis=1)
        q = h_bar + (jnp.dot(cat, Wv.T) + bv)
        dx, dy = nx - cx, ny - cy
        r = jnp.sqrt(dx * dx + dy * dy)
        sel = colf == i
        logp_acc = jnp.where(sel, slp[:, None], logp_acc)
        idx_acc = jnp.where(sel, idx[:, None], idx_acc)
        R_acc = jnp.where(sel, r[:, None], R_acc)
        return q, mask, nx, ny, logp_acc, idx_acc, R_acc

    carry = (q, mask, nx0, ny0, logp_acc, idx_acc, R_acc)
    q, mask, lx, ly, logp_acc, idx_acc, R_acc = jax.lax.fori_loop(1, _S, body, carry)

    logp_ref[:] = logp_acc
    idx_ref[:] = idx_acc
    R_ref[:] = R_acc
    last_ref[:] = jnp.concatenate([lx[:, None], ly[:, None]], axis=1)


def kernel(low_context_vector, original_node, mask, id, low_init_w, W_ctx,
           b_ctx, W_v, b_v, W_t, b_t, W_q, b_q, v_ptr):
    B, S, D, H = _B, _S, _D, _H
    skey = jax.random.key(42)
    gum = jnp.stack([
        jax.random.gumbel(jax.random.fold_in(skey, i), (B, S), jnp.float32)
        for i in range(S)
    ])
    f0 = (jnp.asarray(id) == 0).astype(jnp.int32).reshape(1)
    nodex = original_node[:, :, 0]
    nodey = original_node[:, :, 1]

    out_shapes = (
        jax.ShapeDtypeStruct((B, S), jnp.float32),
        jax.ShapeDtypeStruct((B, S), jnp.int32),
        jax.ShapeDtypeStruct((B, 2), jnp.float32),
        jax.ShapeDtypeStruct((B, S), jnp.float32),
    )
    vmem = pl.BlockSpec(memory_space=pltpu.VMEM)
    smem = pl.BlockSpec(memory_space=pltpu.SMEM)
    logp, idx, last, R = pl.pallas_call(
        _decoder_kernel,
        out_shape=out_shapes,
        in_specs=[smem] + [vmem] * 15,
        out_specs=(vmem, vmem, vmem, vmem),
    )(f0, low_context_vector, nodex, nodey, mask, gum,
      low_init_w.reshape(1, 2 * D), W_ctx, b_ctx, W_v, b_v,
      W_t, b_t, W_q, b_q, v_ptr.reshape(H, 1))

    init_node = original_node[:, 0:1, :]
    return (logp, idx, init_node, last.reshape(B, 1, 2), R)


# speedup vs baseline: 2.5926x; 1.1429x over previous
import jax
import jax.numpy as jnp
from jax.experimental import pallas as pl
from jax.experimental.pallas import tpu as pltpu

_B, _S, _D, _H = 128, 32, 128, 128
_C = 10.0
_NEG = -jnp.inf


def _decoder_kernel(f0_ref, lcv_ref, nodex_ref, nodey_ref, mask_ref, gum_ref,
                    liw_ref, Wc_ref, bc_ref, Wv_ref, bv_ref,
                    Wt_ref, bt_ref, Wq_ref, bq_ref, vp_ref,
                    logp_ref, idx_ref, last_ref, R_ref,
                    T_ref, qh_ref, L_ref):
    lcv = lcv_ref[:]
    Wv = Wv_ref[:]
    Wq = Wq_ref[:]
    vp = vp_ref[:]
    bq = bq_ref[:]
    bv = bv_ref[:]

    lcv2 = lcv.reshape(_B * _S, _D)
    T = (jnp.dot(lcv2, Wt_ref[:].T) + bt_ref[:]).reshape(_B, _S, _H)
    T_ref[:] = T
    h_bar = jnp.dot(jnp.mean(lcv, axis=1), Wc_ref[:].T) + bc_ref[:]
    q0 = h_bar + (jnp.dot(liw_ref[:], Wv.T) + bv)

    col = jax.lax.broadcasted_iota(jnp.int32, (_B, _S), 1)

    def softmax_sample(lg, mask, g):
        lg = jnp.where(mask == 1.0, _NEG, lg)
        shifted = lg - jnp.max(lg, axis=1, keepdims=True)
        logp = shifted - jnp.log(jnp.sum(jnp.exp(shifted), axis=1, keepdims=True))
        score = lg + g
        smax = jnp.max(score, axis=1, keepdims=True)
        idx = jnp.min(jnp.where(score == smax, col, _S), axis=1)
        return idx, logp

    mask = mask_ref[:]
    qh0 = jnp.dot(q0, Wq.T) + bq
    u0 = jnp.tanh(T + qh0[:, None, :])
    lg0 = _C * jnp.tanh(jnp.dot(u0.reshape(_B * _S, _H), vp).reshape(_B, _S))
    idx0, logp0 = softmax_sample(lg0, mask, gum_ref[0])
    idx0 = jnp.where(f0_ref[0] == 1, jnp.zeros_like(idx0), idx0)
    oh0 = col == idx0[:, None]
    slp0 = jnp.sum(jnp.where(oh0, logp0, 0.0), axis=1)
    mask = jnp.where(oh0, 1.0, mask)
    ohf0 = jnp.where(oh0, 1.0, 0.0)
    ih = jnp.sum(lcv * ohf0[:, :, None], axis=1)
    nx0 = jnp.sum(jnp.where(oh0, nodex_ref[:], 0.0), axis=1)
    ny0 = jnp.sum(jnp.where(oh0, nodey_ref[:], 0.0), axis=1)
    cx, cy = nodex_ref[:, 0], nodey_ref[:, 0]
    dx0, dy0 = nx0 - cx, ny0 - cy
    r0 = jnp.sqrt(dx0 * dx0 + dy0 * dy0)

    cat_all = jnp.concatenate(
        [jnp.broadcast_to(ih[:, None, :], (_B, _S, _D)), lcv], axis=2)
    allq = h_bar[:, None, :] + (
        jnp.dot(cat_all.reshape(_B * _S, 2 * _D), Wv.T) + bv).reshape(_B, _S, _D)
    qh_ref[:] = (jnp.dot(allq.reshape(_B * _S, _D), Wq.T) + bq).reshape(_B, _S, _H)

    def build(p, _):
        qh = qh_ref[:, p, :]
        u = jnp.tanh(T_ref[:] + qh[:, None, :])
        row = _C * jnp.tanh(jnp.dot(u.reshape(_B * _S, _H), vp).reshape(_B, _S))
        L_ref[:, pl.ds(p, 1), :] = row[:, None, :]
        return 0
    jax.lax.fori_loop(0, _S, build, 0, unroll=2)

    logp_acc = jnp.where(col == 0, slp0[:, None], 0.0)
    idx_acc = jnp.where(col == 0, idx0[:, None], 0)
    R_acc = jnp.where(col == 0, r0[:, None], 0.0)

    def body(i, carry):
        ohp, mask, cx, cy, logp_acc, idx_acc, R_acc = carry
        lg = jnp.sum(L_ref[:] * ohp[:, :, None], axis=1)
        idx, logp = softmax_sample(lg, mask, gum_ref[i])
        oh = col == idx[:, None]
        slp = jnp.sum(jnp.where(oh, logp, 0.0), axis=1)
        mask = jnp.where(oh, 1.0, mask)
        ohf = jnp.where(oh, 1.0, 0.0)
        nx = jnp.sum(jnp.where(oh, nodex_ref[:], 0.0), axis=1)
        ny = jnp.sum(jnp.where(oh, nodey_ref[:], 0.0), axis=1)
        dx, dy = nx - cx, ny - cy
        r = jnp.sqrt(dx * dx + dy * dy)
        sel = col == i
        logp_acc = jnp.where(sel, slp[:, None], logp_acc)
        idx_acc = jnp.where(sel, idx[:, None], idx_acc)
        R_acc = jnp.where(sel, r[:, None], R_acc)
        return ohf, mask, nx, ny, logp_acc, idx_acc, R_acc

    carry = (ohf0, mask, nx0, ny0, logp_acc, idx_acc, R_acc)
    _, mask, lx, ly, logp_acc, idx_acc, R_acc = jax.lax.fori_loop(
        1, _S, body, carry)

    logp_ref[:] = logp_acc
    idx_ref[:] = idx_acc
    R_ref[:] = R_acc
    last_ref[:] = jnp.concatenate([lx[:, None], ly[:, None]], axis=1)


def kernel(low_context_vector, original_node, mask, id, low_init_w, W_ctx,
           b_ctx, W_v, b_v, W_t, b_t, W_q, b_q, v_ptr):
    B, S, D, H = _B, _S, _D, _H
    skey = jax.random.key(42)
    gum = jnp.stack([
        jax.random.gumbel(jax.random.fold_in(skey, i), (B, S), jnp.float32)
        for i in range(S)
    ])
    f0 = (jnp.asarray(id) == 0).astype(jnp.int32).reshape(1)
    nodex = original_node[:, :, 0]
    nodey = original_node[:, :, 1]

    out_shapes = (
        jax.ShapeDtypeStruct((B, S), jnp.float32),
        jax.ShapeDtypeStruct((B, S), jnp.int32),
        jax.ShapeDtypeStruct((B, 2), jnp.float32),
        jax.ShapeDtypeStruct((B, S), jnp.float32),
    )
    vmem = pl.BlockSpec(memory_space=pltpu.VMEM)
    smem = pl.BlockSpec(memory_space=pltpu.SMEM)
    logp, idx, last, R = pl.pallas_call(
        _decoder_kernel,
        out_shape=out_shapes,
        in_specs=[smem] + [vmem] * 15,
        out_specs=(vmem, vmem, vmem, vmem),
        scratch_shapes=[
            pltpu.VMEM((B, S, H), jnp.float32),
            pltpu.VMEM((B, S, H), jnp.float32),
            pltpu.VMEM((B, S, S), jnp.float32),
        ],
    )(f0, low_context_vector, nodex, nodey, mask, gum,
      low_init_w.reshape(1, 2 * D), W_ctx, b_ctx, W_v, b_v,
      W_t, b_t, W_q, b_q, v_ptr.reshape(H, 1))

    init_node = original_node[:, 0:1, :]
    return (logp, idx, init_node, last.reshape(B, 1, 2), R)


# speedup vs baseline: 6.9789x; 2.6918x over previous
import jax
import jax.numpy as jnp
from jax.experimental import pallas as pl
from jax.experimental.pallas import tpu as pltpu

_B, _S, _D, _H = 128, 32, 128, 128
_C = 10.0
_NEG = -jnp.inf


def _decoder_kernel(f0_ref, lcv_ref, nodex_ref, nodey_ref, mask_ref, gum_ref,
                    liw_ref, Wc_ref, bc_ref, Wv_ref, bv_ref,
                    Wt_ref, bt_ref, Wq_ref, bq_ref, vp_ref,
                    logp_ref, idx_ref, last_ref, R_ref,
                    T_ref, qh_ref, L_ref):
    lcv = lcv_ref[:]
    Wv = Wv_ref[:]
    Wq = Wq_ref[:]
    vp = vp_ref[:]
    bq = bq_ref[:]
    bv = bv_ref[:]

    lcv2 = lcv.reshape(_B * _S, _D)
    T = (jnp.dot(lcv2, Wt_ref[:].T) + bt_ref[:]).reshape(_B, _S, _H)
    T_ref[:] = T
    h_bar = jnp.dot(jnp.mean(lcv, axis=1), Wc_ref[:].T) + bc_ref[:]
    q0 = h_bar + (jnp.dot(liw_ref[:], Wv.T) + bv)

    col = jax.lax.broadcasted_iota(jnp.int32, (_B, _S), 1)

    def softmax_sample(lg, mask, g):
        lg = jnp.where(mask == 1.0, _NEG, lg)
        shifted = lg - jnp.max(lg, axis=1, keepdims=True)
        logp = shifted - jnp.log(jnp.sum(jnp.exp(shifted), axis=1, keepdims=True))
        score = lg + g
        smax = jnp.max(score, axis=1, keepdims=True)
        idx = jnp.min(jnp.where(score == smax, col, _S), axis=1)
        return idx, logp

    mask = mask_ref[:]
    qh0 = jnp.dot(q0, Wq.T) + bq
    u0 = jnp.tanh(T + qh0[:, None, :])
    lg0 = _C * jnp.tanh(jnp.dot(u0.reshape(_B * _S, _H), vp).reshape(_B, _S))
    idx0, logp0 = softmax_sample(lg0, mask, gum_ref[0])
    idx0 = jnp.where(f0_ref[0] == 1, jnp.zeros_like(idx0), idx0)
    oh0 = col == idx0[:, None]
    slp0 = jnp.sum(jnp.where(oh0, logp0, 0.0), axis=1)
    mask = jnp.where(oh0, 1.0, mask)
    ohf0 = jnp.where(oh0, 1.0, 0.0)
    ih = jnp.sum(lcv * ohf0[:, :, None], axis=1)
    nx0 = jnp.sum(jnp.where(oh0, nodex_ref[:], 0.0), axis=1)
    ny0 = jnp.sum(jnp.where(oh0, nodey_ref[:], 0.0), axis=1)
    cx, cy = nodex_ref[:, 0], nodey_ref[:, 0]
    dx0, dy0 = nx0 - cx, ny0 - cy
    r0 = jnp.sqrt(dx0 * dx0 + dy0 * dy0)

    cat_all = jnp.concatenate(
        [jnp.broadcast_to(ih[:, None, :], (_B, _S, _D)), lcv], axis=2)
    allq = h_bar[:, None, :] + (
        jnp.dot(cat_all.reshape(_B * _S, 2 * _D), Wv.T) + bv).reshape(_B, _S, _D)
    qh_ref[:] = (jnp.dot(allq.reshape(_B * _S, _D), Wq.T) + bq).reshape(_B, _S, _H)

    def build(p, _):
        qh = qh_ref[:, p, :]
        u = jnp.tanh(T_ref[:] + qh[:, None, :])
        row = _C * jnp.tanh(jnp.dot(u.reshape(_B * _S, _H), vp).reshape(_B, _S))
        L_ref[:, pl.ds(p, 1), :] = row[:, None, :]
        return 0
    jax.lax.fori_loop(0, _S, build, 0, unroll=2)

    logp_acc = jnp.where(col == 0, slp0[:, None], 0.0)
    idx_acc = jnp.where(col == 0, idx0[:, None], 0)
    R_acc = jnp.where(col == 0, r0[:, None], 0.0)

    def body(i, carry):
        ohp, mask, cx, cy, logp_acc, idx_acc, R_acc = carry
        lg = jnp.sum(L_ref[:] * ohp[:, :, None], axis=1)
        idx, logp = softmax_sample(lg, mask, gum_ref[i])
        oh = col == idx[:, None]
        slp = jnp.sum(jnp.where(oh, logp, 0.0), axis=1)
        mask = jnp.where(oh, 1.0, mask)
        ohf = jnp.where(oh, 1.0, 0.0)
        nx = jnp.sum(jnp.where(oh, nodex_ref[:], 0.0), axis=1)
        ny = jnp.sum(jnp.where(oh, nodey_ref[:], 0.0), axis=1)
        dx, dy = nx - cx, ny - cy
        r = jnp.sqrt(dx * dx + dy * dy)
        sel = col == i
        logp_acc = jnp.where(sel, slp[:, None], logp_acc)
        idx_acc = jnp.where(sel, idx[:, None], idx_acc)
        R_acc = jnp.where(sel, r[:, None], R_acc)
        return ohf, mask, nx, ny, logp_acc, idx_acc, R_acc

    carry = (ohf0, mask, nx0, ny0, logp_acc, idx_acc, R_acc)
    _, mask, lx, ly, logp_acc, idx_acc, R_acc = jax.lax.fori_loop(
        1, _S, body, carry)

    logp_ref[:] = logp_acc
    idx_ref[:] = idx_acc
    R_ref[:] = R_acc
    last_ref[:] = jnp.concatenate([lx[:, None], ly[:, None]], axis=1)


def kernel(low_context_vector, original_node, mask, id, low_init_w, W_ctx,
           b_ctx, W_v, b_v, W_t, b_t, W_q, b_q, v_ptr):
    B, S, D, H = _B, _S, _D, _H
    with jax.ensure_compile_time_eval():
        skey = jax.random.key(42)
        gum = jnp.stack([
            jax.random.gumbel(jax.random.fold_in(skey, i), (B, S), jnp.float32)
            for i in range(S)
        ])
    f0 = (jnp.asarray(id) == 0).astype(jnp.int32).reshape(1)
    nodex = original_node[:, :, 0]
    nodey = original_node[:, :, 1]

    out_shapes = (
        jax.ShapeDtypeStruct((B, S), jnp.float32),
        jax.ShapeDtypeStruct((B, S), jnp.int32),
        jax.ShapeDtypeStruct((B, 2), jnp.float32),
        jax.ShapeDtypeStruct((B, S), jnp.float32),
    )
    vmem = pl.BlockSpec(memory_space=pltpu.VMEM)
    smem = pl.BlockSpec(memory_space=pltpu.SMEM)
    logp, idx, last, R = pl.pallas_call(
        _decoder_kernel,
        out_shape=out_shapes,
        in_specs=[smem] + [vmem] * 15,
        out_specs=(vmem, vmem, vmem, vmem),
        scratch_shapes=[
            pltpu.VMEM((B, S, H), jnp.float32),
            pltpu.VMEM((B, S, H), jnp.float32),
            pltpu.VMEM((B, S, S), jnp.float32),
        ],
    )(f0, low_context_vector, nodex, nodey, mask, gum,
      low_init_w.reshape(1, 2 * D), W_ctx, b_ctx, W_v, b_v,
      W_t, b_t, W_q, b_q, v_ptr.reshape(H, 1))

    init_node = original_node[:, 0:1, :]
    return (logp, idx, init_node, last.reshape(B, 1, 2), R)


# speedup vs baseline: 11.0642x; 1.5854x over previous
import jax
import jax.numpy as jnp
from jax.experimental import pallas as pl
from jax.experimental.pallas import tpu as pltpu

_B, _S, _D, _H = 128, 32, 128, 128
_C = 10.0
_NEG = -jnp.inf


def _decoder_kernel(f0_ref, lcvt_ref, nodext_ref, nodeyt_ref, maskt_ref,
                    gum_ref, liw_ref, Wc_ref, bc_ref, Wv_ref, bv_ref,
                    Wt_ref, bt_ref, Wq_ref, bq_ref, vp_ref,
                    logp_ref, idx_ref, last_ref, R_ref,
                    T_ref, qh_ref, L_ref):
    lcvt = lcvt_ref[:]
    Wv = Wv_ref[:]
    Wq = Wq_ref[:]
    vp = vp_ref[:]
    bq = bq_ref[:]
    bv = bv_ref[:]

    lcv2 = lcvt.reshape(_S * _B, _D)
    T = (jnp.dot(lcv2, Wt_ref[:].T) + bt_ref[:]).reshape(_S, _B, _H)
    T_ref[:] = T
    h_bar = jnp.dot(jnp.mean(lcvt, axis=0), Wc_ref[:].T) + bc_ref[:]
    q0 = h_bar + (jnp.dot(liw_ref[:], Wv.T) + bv)

    row = jax.lax.broadcasted_iota(jnp.int32, (_S, _B), 0)

    def softmax_sample(lg, mask, g):
        lg = jnp.where(mask == 1.0, _NEG, lg)
        shifted = lg - jnp.max(lg, axis=0, keepdims=True)
        logp = shifted - jnp.log(jnp.sum(jnp.exp(shifted), axis=0, keepdims=True))
        score = lg + g
        smax = jnp.max(score, axis=0, keepdims=True)
        idx = jnp.min(jnp.where(score == smax, row, _S), axis=0)
        return idx, logp

    mask = maskt_ref[:]
    qh0 = jnp.dot(q0, Wq.T) + bq
    u0 = jnp.tanh(T + qh0[None, :, :])
    lg0 = _C * jnp.tanh(jnp.dot(u0.reshape(_S * _B, _H), vp).reshape(_S, _B))
    idx0, logp0 = softmax_sample(lg0, mask, gum_ref[0])
    idx0 = jnp.where(f0_ref[0] == 1, jnp.zeros_like(idx0), idx0)
    oh0 = row == idx0[None, :]
    slp0 = jnp.sum(jnp.where(oh0, logp0, 0.0), axis=0)
    mask = jnp.where(oh0, 1.0, mask)
    ohf0 = jnp.where(oh0, 1.0, 0.0)
    ih = jnp.sum(lcvt * ohf0[:, :, None], axis=0)
    nx0 = jnp.sum(jnp.where(oh0, nodext_ref[:], 0.0), axis=0)
    ny0 = jnp.sum(jnp.where(oh0, nodeyt_ref[:], 0.0), axis=0)
    cx, cy = nodext_ref[0], nodeyt_ref[0]
    dx0, dy0 = nx0 - cx, ny0 - cy
    r0 = jnp.sqrt(dx0 * dx0 + dy0 * dy0)

    cat_all = jnp.concatenate(
        [jnp.broadcast_to(ih[None, :, :], (_S, _B, _D)), lcvt], axis=2)
    allq = h_bar[None, :, :] + (
        jnp.dot(cat_all.reshape(_S * _B, 2 * _D), Wv.T) + bv).reshape(_S, _B, _D)
    qh_ref[:] = (jnp.dot(allq.reshape(_S * _B, _D), Wq.T) + bq).reshape(_S, _B, _H)

    def build(p, _):
        qh = qh_ref[p]
        u = jnp.tanh(T_ref[:] + qh[None, :, :])
        lrow = _C * jnp.tanh(jnp.dot(u.reshape(_S * _B, _H), vp).reshape(_S, _B))
        L_ref[pl.ds(p, 1)] = lrow[None]
        return 0
    jax.lax.fori_loop(0, _S, build, 0, unroll=2)

    logp_acc = jnp.where(row == 0, slp0[None, :], 0.0)
    idx_acc = jnp.where(row == 0, idx0[None, :], 0)
    R_acc = jnp.where(row == 0, r0[None, :], 0.0)

    def body(i, carry):
        ohp, mask, cx, cy, logp_acc, idx_acc, R_acc = carry
        lg = jnp.sum(L_ref[:] * ohp[:, None, :], axis=0)
        idx, logp = softmax_sample(lg, mask, gum_ref[i])
        oh = row == idx[None, :]
        slp = jnp.sum(jnp.where(oh, logp, 0.0), axis=0)
        mask = jnp.where(oh, 1.0, mask)
        ohf = jnp.where(oh, 1.0, 0.0)
        nx = jnp.sum(jnp.where(oh, nodext_ref[:], 0.0), axis=0)
        ny = jnp.sum(jnp.where(oh, nodeyt_ref[:], 0.0), axis=0)
        dx, dy = nx - cx, ny - cy
        r = jnp.sqrt(dx * dx + dy * dy)
        sel = row == i
        logp_acc = jnp.where(sel, slp[None, :], logp_acc)
        idx_acc = jnp.where(sel, idx[None, :], idx_acc)
        R_acc = jnp.where(sel, r[None, :], R_acc)
        return ohf, mask, nx, ny, logp_acc, idx_acc, R_acc

    carry = (ohf0, mask, nx0, ny0, logp_acc, idx_acc, R_acc)
    _, mask, lx, ly, logp_acc, idx_acc, R_acc = jax.lax.fori_loop(
        1, _S, body, carry)

    logp_ref[:] = logp_acc.T
    idx_ref[:] = idx_acc.T
    R_ref[:] = R_acc.T
    last_ref[:] = jnp.concatenate([lx[:, None], ly[:, None]], axis=1)


def kernel(low_context_vector, original_node, mask, id, low_init_w, W_ctx,
           b_ctx, W_v, b_v, W_t, b_t, W_q, b_q, v_ptr):
    B, S, D, H = _B, _S, _D, _H
    with jax.ensure_compile_time_eval():
        skey = jax.random.key(42)
        gum = jnp.stack([
            jax.random.gumbel(jax.random.fold_in(skey, i), (B, S), jnp.float32).T
            for i in range(S)
        ])
    f0 = (jnp.asarray(id) == 0).astype(jnp.int32).reshape(1)
    lcvt = jnp.transpose(low_context_vector, (1, 0, 2))
    node_t = jnp.transpose(original_node, (2, 1, 0))
    nodext, nodeyt = node_t[0], node_t[1]
    maskt = mask.T

    out_shapes = (
        jax.ShapeDtypeStruct((B, S), jnp.float32),
        jax.ShapeDtypeStruct((B, S), jnp.int32),
        jax.ShapeDtypeStruct((B, 2), jnp.float32),
        jax.ShapeDtypeStruct((B, S), jnp.float32),
    )
    vmem = pl.BlockSpec(memory_space=pltpu.VMEM)
    smem = pl.BlockSpec(memory_space=pltpu.SMEM)
    logp, idx, last, R = pl.pallas_call(
        _decoder_kernel,
        out_shape=out_shapes,
        in_specs=[smem] + [vmem] * 15,
        out_specs=(vmem, vmem, vmem, vmem),
        scratch_shapes=[
            pltpu.VMEM((S, B, H), jnp.float32),
            pltpu.VMEM((S, B, H), jnp.float32),
            pltpu.VMEM((S, S, B), jnp.float32),
        ],
    )(f0, lcvt, nodext, nodeyt, maskt, gum,
      low_init_w.reshape(1, 2 * D), W_ctx, b_ctx, W_v, b_v,
      W_t, b_t, W_q, b_q, v_ptr.reshape(H, 1))

    init_node = original_node[:, 0:1, :]
    return (logp, idx, init_node, last.reshape(B, 1, 2), R)
